# Initial kernel scaffold; baseline (speedup 1.0000x reference)
import jax
import jax.numpy as jnp
from jax.experimental import pallas as pl


def kernel(x, edge_index, gcn1_W, gcn1_b, gat1_W, gat1_att_src, gat1_att_dst, gat1_b, gcn2_W, gcn2_b, gat2_W, gat2_att_src, gat2_att_dst, gat2_b, out_W, out_b):
    raise NotImplementedError("write your pallas kernel here")


# speedup vs baseline: 1.2024x; 1.2024x over previous
import functools

import jax
import jax.numpy as jnp
from jax.experimental import pallas as pl
from jax.experimental.pallas import tpu as pltpu

NN = 10000
DD = 128
CC = 64
NEG_SLOPE = 0.2


def _mm_kernel(x_ref, w_ref, o_ref):
    o_ref[...] = jnp.dot(x_ref[...], w_ref[...],
                         preferred_element_type=jnp.float32)


def tc_matmul(x, w):
    return pl.pallas_call(
        _mm_kernel,
        out_shape=jax.ShapeDtypeStruct((x.shape[0], w.shape[1]), jnp.float32),
    )(x, w)


def _dis_kernel(deg_ref, o_ref):
    d = deg_ref[...]
    o_ref[...] = jnp.where(d > 0, jax.lax.rsqrt(d), 0.0)


def tc_dis(deg):
    return pl.pallas_call(
        _dis_kernel,
        out_shape=jax.ShapeDtypeStruct(deg.shape, jnp.float32),
    )(deg)


def _post_gcn_prep_kernel(acc_ref, b_ref, w_ref, o_ref):
    h = jnp.maximum(acc_ref[...] + b_ref[...][None, :], 0.0)
    o_ref[...] = jnp.dot(h, w_ref[...], preferred_element_type=jnp.float32)


def tc_post_gcn_prep(acc, b, w):
    return pl.pallas_call(
        _post_gcn_prep_kernel,
        out_shape=jax.ShapeDtypeStruct((acc.shape[0], w.shape[1]), jnp.float32),
    )(acc, b, w)


def _att_kernel(xw_ref, asrc_ref, adst_ref, as_ref, ad_ref, u_ref):
    xw = xw_ref[...]
    a_s = jnp.sum(xw * asrc_ref[...][None, :], axis=1)
    a_d = jnp.sum(xw * adst_ref[...][None, :], axis=1)
    as_ref[...] = a_s
    ad_ref[...] = a_d
    amax = jnp.max(a_s)
    e_ub = amax + a_d
    u_ref[...] = jnp.where(e_ub >= 0, e_ub, NEG_SLOPE * e_ub)


def tc_att(xw, a_src, a_dst):
    n = xw.shape[0]
    return pl.pallas_call(
        _att_kernel,
        out_shape=[jax.ShapeDtypeStruct((n,), jnp.float32)] * 3,
    )(xw, a_src, a_dst)


def _post_gat_prep_kernel(acc_ref, s_ref, b_ref, w_ref, o_ref):
    s = s_ref[...]
    h = jnp.maximum(acc_ref[...] / (s[:, None] + 1e-16) + b_ref[...][None, :],
                    0.0)
    o_ref[...] = jnp.dot(h, w_ref[...], preferred_element_type=jnp.float32)


def tc_post_gat_prep(acc, s, b, w):
    return pl.pallas_call(
        _post_gat_prep_kernel,
        out_shape=jax.ShapeDtypeStruct((acc.shape[0], w.shape[1]), jnp.float32),
    )(acc, s, b, w)


def _final_kernel(acc_ref, s_ref, b_ref, w_ref, ob_ref, h_ref, z_ref):
    s = s_ref[...]
    h = jnp.maximum(acc_ref[...] / (s[:, None] + 1e-16) + b_ref[...][None, :],
                    0.0)
    h_ref[...] = h
    z_ref[...] = jnp.dot(h, w_ref[...],
                         preferred_element_type=jnp.float32) + ob_ref[...][None, :]


def tc_final(acc, s, b, w, ob):
    n = acc.shape[0]
    return pl.pallas_call(
        _final_kernel,
        out_shape=[jax.ShapeDtypeStruct((n, DD), jnp.float32),
                   jax.ShapeDtypeStruct((n, CC), jnp.float32)],
    )(acc, s, b, w, ob)


def sp_degree(src, dst, e_real):
    deg = jnp.zeros((NN,), jnp.float32).at[dst].add(1.0)
    return deg


def sp_gcn_pass(src, dst, e_real, dis, xw):
    norm = dis[src] * dis[dst]
    return jnp.zeros_like(xw).at[dst].add(norm[:, None] * xw[src])


def sp_gat_pass(src, dst, e_real, a_s, a_d, u, xw):
    e = a_s[src] + a_d[dst]
    e = jnp.where(e >= 0, e, NEG_SLOPE * e)
    ex = jnp.exp(e - u[dst])
    acc = jnp.zeros_like(xw).at[dst].add(ex[:, None] * xw[src])
    s_sum = jnp.zeros((NN,), jnp.float32).at[dst].add(ex)
    return acc, s_sum


def kernel(x, edge_index, gcn1_W, gcn1_b, gat1_W, gat1_att_src, gat1_att_dst,
           gat1_b, gcn2_W, gcn2_b, gat2_W, gat2_att_src, gat2_att_dst,
           gat2_b, out_W, out_b):
    loops = jnp.arange(NN, dtype=edge_index.dtype)
    src = jnp.concatenate([edge_index[0], loops]).astype(jnp.int32)
    dst = jnp.concatenate([edge_index[1], loops]).astype(jnp.int32)
    e_real = src.shape[0]

    deg = sp_degree(src, dst, e_real)
    dis = tc_dis(deg)

    xw1 = tc_matmul(x, gcn1_W)
    acc1 = sp_gcn_pass(src, dst, e_real, dis, xw1)

    xw2 = tc_post_gcn_prep(acc1, gcn1_b, gat1_W)
    as2, ad2, u2 = tc_att(xw2, gat1_att_src, gat1_att_dst)
    acc2, s2 = sp_gat_pass(src, dst, e_real, as2, ad2, u2, xw2)

    xw3 = tc_post_gat_prep(acc2, s2, gat1_b, gcn2_W)
    acc3 = sp_gcn_pass(src, dst, e_real, dis, xw3)

    xw4 = tc_post_gcn_prep(acc3, gcn2_b, gat2_W)
    as4, ad4, u4 = tc_att(xw4, gat2_att_src, gat2_att_dst)
    acc4, s4 = sp_gat_pass(src, dst, e_real, as4, ad4, u4, xw4)

    h, z = tc_final(acc4, s4, gat2_b, out_W, out_b)
    return (h, z)


# speedup vs baseline: 15.8413x; 13.1750x over previous
import functools

import jax
import jax.numpy as jnp
from jax import lax
from jax.experimental import pallas as pl
from jax.experimental.pallas import tpu as pltpu
from jax.experimental.pallas import tpu_sc as plsc

NN = 10000
NP2 = 10240
DD = 128
CC = 64
NEG_SLOPE = 0.2

ER = 330000
NC, NS, LANES = 2, 16, 16
NW = NC * NS
KE = 128
BW = 10368
EPAD = BW * NW
NCH = BW // KE
RPS = NP2 // NS

_sc_mesh = plsc.VectorSubcoreMesh(core_axis_name="c", subcore_axis_name="s")


def _mm_kernel(x_ref, w_ref, o_ref):
    o_ref[...] = jnp.dot(x_ref[...], w_ref[...],
                         preferred_element_type=jnp.float32)


def tc_matmul(x, w):
    return pl.pallas_call(
        _mm_kernel,
        out_shape=jax.ShapeDtypeStruct((x.shape[0], w.shape[1]), jnp.float32),
    )(x, w)


def _dis_kernel(d0_ref, d1_ref, o_ref):
    d = d0_ref[...][:NN] + d1_ref[...][:NN]
    o_ref[...] = jnp.where(d > 0, lax.rsqrt(d), 0.0)


def tc_dis(deg0, deg1):
    return pl.pallas_call(
        _dis_kernel,
        out_shape=jax.ShapeDtypeStruct((NN,), jnp.float32),
    )(deg0, deg1)


def _post_gcn_prep_kernel(a0_ref, a1_ref, b_ref, w_ref, o_ref):
    h = jnp.maximum(a0_ref[...] + a1_ref[...] + b_ref[...][None, :], 0.0)
    o_ref[...] = jnp.dot(h, w_ref[...], preferred_element_type=jnp.float32)


def tc_post_gcn_prep(acc0, acc1, b, w):
    return pl.pallas_call(
        _post_gcn_prep_kernel,
        out_shape=jax.ShapeDtypeStruct((NN, w.shape[1]), jnp.float32),
    )(acc0, acc1, b, w)


def _att_kernel(xw_ref, asrc_ref, adst_ref, as_ref, ad_ref, u_ref):
    xw = xw_ref[...]
    a_s = jnp.sum(xw * asrc_ref[...][None, :], axis=1)
    a_d = jnp.sum(xw * adst_ref[...][None, :], axis=1)
    as_ref[...] = a_s
    ad_ref[...] = a_d
    e_ub = jnp.max(a_s) + a_d
    u_ref[...] = jnp.where(e_ub >= 0, e_ub, NEG_SLOPE * e_ub)


def tc_att(xw, a_src, a_dst):
    return pl.pallas_call(
        _att_kernel,
        out_shape=[jax.ShapeDtypeStruct((NN,), jnp.float32)] * 3,
    )(xw, a_src, a_dst)


def _post_gat_prep_kernel(a0_ref, a1_ref, s0_ref, s1_ref, b_ref, w_ref, o_ref):
    s = (s0_ref[...][:NN] + s1_ref[...][:NN])[:, None] + 1e-16
    h = jnp.maximum((a0_ref[...] + a1_ref[...]) / s + b_ref[...][None, :], 0.0)
    o_ref[...] = jnp.dot(h, w_ref[...], preferred_element_type=jnp.float32)


def tc_post_gat_prep(acc0, acc1, s0, s1, b, w):
    return pl.pallas_call(
        _post_gat_prep_kernel,
        out_shape=jax.ShapeDtypeStruct((NN, w.shape[1]), jnp.float32),
    )(acc0, acc1, s0, s1, b, w)


def _final_kernel(a0_ref, a1_ref, s0_ref, s1_ref, b_ref, w_ref, ob_ref,
                  h_ref, z_ref):
    s = (s0_ref[...][:NN] + s1_ref[...][:NN])[:, None] + 1e-16
    h = jnp.maximum((a0_ref[...] + a1_ref[...]) / s + b_ref[...][None, :], 0.0)
    h_ref[...] = h
    z_ref[...] = (jnp.dot(h, w_ref[...], preferred_element_type=jnp.float32)
                  + ob_ref[...][None, :])


def tc_final(acc0, acc1, s0, s1, b, w, ob):
    return pl.pallas_call(
        _final_kernel,
        out_shape=[jax.ShapeDtypeStruct((NN, DD), jnp.float32),
                   jax.ShapeDtypeStruct((NN, CC), jnp.float32)],
    )(acc0, acc1, s0, s1, b, w, ob)


def _edge_mask(eb, j):
    ids = eb + j * LANES + lax.iota(jnp.int32, LANES)
    return ids < ER


def _deg_body(dst_hbm, z1_hbm, out0, out1, didx, ones_v, deg_sh, sem):
    c = lax.axis_index("c")
    s = lax.axis_index("s")
    wid = c * NS + s
    pltpu.sync_copy(z1_hbm, deg_sh.at[pl.ds(s * RPS, RPS)])
    plsc.subcore_barrier()
    base = wid * BW

    def chunk(i, carry):
        eb = base + i * KE
        pltpu.sync_copy(dst_hbm.at[pl.ds(eb, KE)], didx)

        def grp(j, carry2):
            m = _edge_mask(eb, j)
            ones_v[pl.ds(j * LANES, LANES)] = jnp.where(m, 1.0, 0.0)
            return carry2

        lax.fori_loop(0, KE // LANES, grp, 0)
        pltpu.sync_copy(ones_v, deg_sh.at[didx], add=True)
        return carry

    lax.fori_loop(0, NCH, chunk, 0)
    plsc.subcore_barrier()
    sl = pl.ds(s * RPS, RPS)

    @pl.when(c == 0)
    def _():
        pltpu.sync_copy(deg_sh.at[sl], out0.at[sl])

    @pl.when(c == 1)
    def _():
        pltpu.sync_copy(deg_sh.at[sl], out1.at[sl])


def sc_degree(dst_p, z1):
    f = pl.kernel(
        _deg_body,
        out_type=[jax.ShapeDtypeStruct((NP2,), jnp.float32)] * 2,
        mesh=_sc_mesh,
        compiler_params=pltpu.CompilerParams(needs_layout_passes=False),
        scratch_types=[
            pltpu.VMEM((KE,), jnp.int32),
            pltpu.VMEM((KE,), jnp.float32),
            pltpu.VMEM_SHARED((NP2,), jnp.float32),
            pltpu.SemaphoreType.DMA,
        ],
    )
    return f(dst_p, z1)


def _gcn_body(src_hbm, dst_hbm, dis_hbm, xw_hbm, z2_hbm, out0, out1,
              dis_v, sidx, didx, rows, wv, acc_sh, sem):
    c = lax.axis_index("c")
    s = lax.axis_index("s")
    wid = c * NS + s
    pltpu.sync_copy(z2_hbm, acc_sh.at[pl.ds(s * RPS, RPS)])
    pltpu.sync_copy(dis_hbm, dis_v)
    plsc.subcore_barrier()
    base = wid * BW

    def chunk(i, carry):
        eb = base + i * KE
        pltpu.sync_copy(src_hbm.at[pl.ds(eb, KE)], sidx)
        pltpu.sync_copy(dst_hbm.at[pl.ds(eb, KE)], didx)
        pltpu.async_copy(xw_hbm.at[sidx], rows, sem).wait()

        def grp(j, carry2):
            sv = sidx[pl.ds(j * LANES, LANES)]
            dv = didx[pl.ds(j * LANES, LANES)]
            w = (plsc.load_gather(dis_v, [sv])
                 * plsc.load_gather(dis_v, [dv]))
            w = jnp.where(_edge_mask(eb, j), w, 0.0)
            wv[pl.ds(j * LANES, LANES)] = w
            return carry2

        lax.fori_loop(0, KE // LANES, grp, 0)

        def scale(e, carry2):
            w = plsc.load_gather(wv, [jnp.full((LANES,), e, jnp.int32)])
            for jj in range(DD // LANES):
                sl = pl.ds(jj * LANES, LANES)
                rows[e, sl] = rows[e, sl] * w
            return carry2

        lax.fori_loop(0, KE, scale, 0)
        pltpu.sync_copy(rows, acc_sh.at[didx], add=True)
        return carry

    lax.fori_loop(0, NCH, chunk, 0)
    plsc.subcore_barrier()
    sl = pl.ds(s * RPS, RPS)

    @pl.when(c == 0)
    def _():
        pltpu.sync_copy(acc_sh.at[sl], out0.at[sl])

    @pl.when(c == 1)
    def _():
        pltpu.sync_copy(acc_sh.at[sl], out1.at[sl])


def sc_gcn_pass(src_p, dst_p, dis, xw, z2):
    f = pl.kernel(
        _gcn_body,
        out_type=[jax.ShapeDtypeStruct((NP2, DD), jnp.float32)] * 2,
        mesh=_sc_mesh,
        compiler_params=pltpu.CompilerParams(needs_layout_passes=False),
        scratch_types=[
            pltpu.VMEM((NN,), jnp.float32),
            pltpu.VMEM((KE,), jnp.int32),
            pltpu.VMEM((KE,), jnp.int32),
            pltpu.VMEM((KE, DD), jnp.float32),
            pltpu.VMEM((KE,), jnp.float32),
            pltpu.VMEM_SHARED((NP2, DD), jnp.float32),
            pltpu.SemaphoreType.DMA,
        ],
    )
    return f(src_p, dst_p, dis, xw, z2)


def _gat_body(src_hbm, dst_hbm, as_hbm, ad_hbm, u_hbm, xw_hbm, z2_hbm, z1_hbm,
              out0, out1, so0, so1,
              as_v, ad_v, u_v, sidx, didx, rows, wv, acc_sh, s_sh, sem):
    c = lax.axis_index("c")
    s = lax.axis_index("s")
    wid = c * NS + s
    pltpu.sync_copy(z2_hbm, acc_sh.at[pl.ds(s * RPS, RPS)])
    pltpu.sync_copy(z1_hbm, s_sh.at[pl.ds(s * RPS, RPS)])
    pltpu.sync_copy(as_hbm, as_v)
    pltpu.sync_copy(ad_hbm, ad_v)
    pltpu.sync_copy(u_hbm, u_v)
    plsc.subcore_barrier()
    base = wid * BW

    def chunk(i, carry):
        eb = base + i * KE
        pltpu.sync_copy(src_hbm.at[pl.ds(eb, KE)], sidx)
        pltpu.sync_copy(dst_hbm.at[pl.ds(eb, KE)], didx)
        pltpu.async_copy(xw_hbm.at[sidx], rows, sem).wait()

        def grp(j, carry2):
            sv = sidx[pl.ds(j * LANES, LANES)]
            dv = didx[pl.ds(j * LANES, LANES)]
            e = plsc.load_gather(as_v, [sv]) + plsc.load_gather(ad_v, [dv])
            e = jnp.where(e >= 0, e, NEG_SLOPE * e)
            ex = jnp.exp(e - plsc.load_gather(u_v, [dv]))
            ex = jnp.where(_edge_mask(eb, j), ex, 0.0)
            wv[pl.ds(j * LANES, LANES)] = ex
            return carry2

        lax.fori_loop(0, KE // LANES, grp, 0)

        def scale(e, carry2):
            w = plsc.load_gather(wv, [jnp.full((LANES,), e, jnp.int32)])
            for jj in range(DD // LANES):
                sl = pl.ds(jj * LANES, LANES)
                rows[e, sl] = rows[e, sl] * w
            return carry2

        lax.fori_loop(0, KE, scale, 0)
        pltpu.sync_copy(rows, acc_sh.at[didx], add=True)
        pltpu.sync_copy(wv, s_sh.at[didx], add=True)
        return carry

    lax.fori_loop(0, NCH, chunk, 0)
    plsc.subcore_barrier()
    sl = pl.ds(s * RPS, RPS)

    @pl.when(c == 0)
    def _():
        pltpu.sync_copy(acc_sh.at[sl], out0.at[sl])
        pltpu.sync_copy(s_sh.at[sl], so0.at[sl])

    @pl.when(c == 1)
    def _():
        pltpu.sync_copy(acc_sh.at[sl], out1.at[sl])
        pltpu.sync_copy(s_sh.at[sl], so1.at[sl])


def sc_gat_pass(src_p, dst_p, a_s, a_d, u, xw, z2, z1):
    f = pl.kernel(
        _gat_body,
        out_type=[jax.ShapeDtypeStruct((NP2, DD), jnp.float32)] * 2
        + [jax.ShapeDtypeStruct((NP2,), jnp.float32)] * 2,
        mesh=_sc_mesh,
        compiler_params=pltpu.CompilerParams(needs_layout_passes=False),
        scratch_types=[
            pltpu.VMEM((NN,), jnp.float32),
            pltpu.VMEM((NN,), jnp.float32),
            pltpu.VMEM((NN,), jnp.float32),
            pltpu.VMEM((KE,), jnp.int32),
            pltpu.VMEM((KE,), jnp.int32),
            pltpu.VMEM((KE, DD), jnp.float32),
            pltpu.VMEM((KE,), jnp.float32),
            pltpu.VMEM_SHARED((NP2, DD), jnp.float32),
            pltpu.VMEM_SHARED((NP2,), jnp.float32),
            pltpu.SemaphoreType.DMA,
        ],
    )
    return f(src_p, dst_p, a_s, a_d, u, xw, z2, z1)


def kernel(x, edge_index, gcn1_W, gcn1_b, gat1_W, gat1_att_src, gat1_att_dst,
           gat1_b, gcn2_W, gcn2_b, gat2_W, gat2_att_src, gat2_att_dst,
           gat2_b, out_W, out_b):
    loops = jnp.arange(NN, dtype=jnp.int32)
    pad = jnp.zeros((EPAD - ER,), jnp.int32)
    src_p = jnp.concatenate([edge_index[0].astype(jnp.int32), loops, pad])
    dst_p = jnp.concatenate([edge_index[1].astype(jnp.int32), loops, pad])
    z1 = jnp.zeros((RPS,), jnp.float32)
    z2 = jnp.zeros((RPS, DD), jnp.float32)

    deg0, deg1 = sc_degree(dst_p, z1)
    dis = tc_dis(deg0, deg1)

    xw1 = tc_matmul(x, gcn1_W)
    a1_0, a1_1 = sc_gcn_pass(src_p, dst_p, dis, xw1, z2)

    xw2 = tc_post_gcn_prep(a1_0[:NN], a1_1[:NN], gcn1_b, gat1_W)
    as2, ad2, u2 = tc_att(xw2, gat1_att_src, gat1_att_dst)
    a2_0, a2_1, s2_0, s2_1 = sc_gat_pass(src_p, dst_p, as2, ad2, u2, xw2,
                                         z2, z1)

    xw3 = tc_post_gat_prep(a2_0[:NN], a2_1[:NN], s2_0, s2_1, gat1_b, gcn2_W)
    a3_0, a3_1 = sc_gcn_pass(src_p, dst_p, dis, xw3, z2)

    xw4 = tc_post_gcn_prep(a3_0[:NN], a3_1[:NN], gcn2_b, gat2_W)
    as4, ad4, u4 = tc_att(xw4, gat2_att_src, gat2_att_dst)
    a4_0, a4_1, s4_0, s4_1 = sc_gat_pass(src_p, dst_p, as4, ad4, u4, xw4,
                                         z2, z1)

    h, z = tc_final(a4_0[:NN], a4_1[:NN], s4_0, s4_1, gat2_b, out_W, out_b)
    return (h, z)
